# baseline (speedup 1.0000x reference)
import jax
import jax.numpy as jnp
import numpy as np
from jax.experimental import pallas as pl

_N = 1024
_C = 8
_GAMMA = 1.0
_PER = 0.001
_SIG1 = float(1.0 / (1.0 + np.exp(-1.0)))


def _loss_body(preds_ref, lab_ref, maskf_ref, a_ref, diag_ref, out_ref):
    preds = preds_ref[...]
    labels = lab_ref[...]
    maskf = maskf_ref[...]
    a_bool = a_ref[...]
    diag_col = diag_ref[...]

    mx = jnp.max(preds, axis=1, keepdims=True)
    lse = jnp.log(jnp.sum(jnp.exp(preds - mx), axis=1, keepdims=True)) + mx
    logp = preds - lse
    cls_iota = jax.lax.broadcasted_iota(jnp.int32, (_N, _C), 1)
    lab_oh = (cls_iota == labels).astype(jnp.float32)
    ce = -jnp.sum(logp * lab_oh) * (1.0 / _N)

    m_cls = lab_oh * maskf
    ncnt = jnp.sum(m_cls, axis=0, keepdims=True)

    a_bf = a_bool.astype(jnp.bfloat16)
    a_f = a_bool.astype(jnp.float32)
    deg_row = jnp.sum(a_f, axis=0, keepdims=True)
    inter = jax.lax.dot_general(a_bf, a_bf, (((1,), (0,)), ((), ())),
                                preferred_element_type=jnp.float32)
    sub_t = deg_row - inter - a_f * (1.0 - diag_col)

    ratio = (1.0 + _SIG1 * sub_t) / (1.0 + _SIG1 * inter)
    v_t = 1.0 / (1.0 + jnp.exp(ratio))

    eg = jnp.exp(_GAMMA * preds)
    u = m_cls / eg
    t = jax.lax.dot_general(v_t, u, (((1,), (0,)), ((), ())),
                            preferred_element_type=jnp.float32)
    p_t = jax.lax.dot_general(m_cls, t * eg, (((0,), (0,)), ((), ())),
                              preferred_element_type=jnp.float32)

    sub_pos = (sub_t > 0.0).astype(jnp.float32)
    inter_pos = (inter > 0.0).astype(jnp.float32)
    s_sub_t = jax.lax.dot_general(
        m_cls,
        jax.lax.dot_general(sub_pos, m_cls, (((1,), (0,)), ((), ())),
                            preferred_element_type=jnp.float32),
        (((0,), (0,)), ((), ())), preferred_element_type=jnp.float32)
    s_inter_t = jax.lax.dot_general(
        m_cls,
        jax.lax.dot_general(inter_pos, m_cls, (((1,), (0,)), ((), ())),
                            preferred_element_type=jnp.float32),
        (((0,), (0,)), ((), ())), preferred_element_type=jnp.float32)

    denom = jnp.reshape(ncnt, (_C, 1)) * ncnt
    recip = jnp.where(denom > 0.0, 1.0 / jnp.where(denom > 0.0, denom, 1.0), 0.0)
    ii = jax.lax.broadcasted_iota(jnp.int32, (_C, _C), 0)
    jj = jax.lax.broadcasted_iota(jnp.int32, (_C, _C), 1)
    keep = jnp.logical_and(jnp.logical_and(s_sub_t > 0.0, s_inter_t > 0.0),
                           ii != jj)
    pair_loss = jnp.sum(jnp.where(keep, p_t * recip, 0.0))

    out_ref[...] = jnp.reshape(ce + _PER * pair_loss, (1, 1))


def kernel(preds, labels, mask, w_values_dict, adj_matrix):
    del w_values_dict
    adj_b = adj_matrix.astype(bool)
    diag_col = jnp.diagonal(adj_b).astype(jnp.float32).reshape(_N, 1)
    lab = labels.astype(jnp.int32).reshape(_N, 1)
    maskf = mask.astype(jnp.float32).reshape(_N, 1)
    out = pl.pallas_call(
        _loss_body,
        out_shape=jax.ShapeDtypeStruct((1, 1), jnp.float32),
    )(preds.astype(jnp.float32), lab, maskf, adj_b, diag_col)
    return out[0, 0]
